# Initial kernel scaffold; baseline (speedup 1.0000x reference)
import jax
import jax.numpy as jnp
from jax.experimental import pallas as pl


def kernel(x, table):
    raise NotImplementedError("write your pallas kernel here")


# speedup vs baseline: 1.3067x; 1.3067x over previous
import functools

import jax
import jax.numpy as jnp
from jax import lax
from jax.experimental import pallas as pl
from jax.experimental.pallas import tpu as pltpu
from jax.experimental.pallas import tpu_sc as plsc

EMB = 32
B = 4096
L = 200
TOTAL = B * L
NC = 2
NS = 16
NW = NC * NS
PER_W = TOTAL // NW
CHUNK = 128
NCHUNK = PER_W // CHUNK

_mesh = plsc.VectorSubcoreMesh(core_axis_name="c", subcore_axis_name="s")


@functools.partial(
    pl.kernel,
    out_type=jax.ShapeDtypeStruct((TOTAL, EMB), jnp.float32),
    mesh=_mesh,
    compiler_params=pltpu.CompilerParams(use_tc_tiling_on_sc=False),
    scratch_types=[
        pltpu.VMEM((NCHUNK, CHUNK), jnp.int32),
        pltpu.VMEM((CHUNK, EMB), jnp.float32),
        pltpu.SemaphoreType.DMA,
    ],
)
def _emb_lookup(x_hbm, table_hbm, out_hbm, idx_v, rows_v, sem):
    wid = lax.axis_index("s") * NC + lax.axis_index("c")
    base = wid * PER_W
    pltpu.sync_copy(x_hbm.at[wid], idx_v)

    def body(j, _):
        pltpu.async_copy(table_hbm.at[idx_v.at[j]], rows_v, sem).wait()
        pltpu.sync_copy(rows_v, out_hbm.at[pl.ds(base + j * CHUNK, CHUNK)])
        return 0

    lax.fori_loop(0, NCHUNK, body, 0)


def kernel(x, table):
    xw = x.reshape(NW, NCHUNK, CHUNK).astype(jnp.int32)
    out = _emb_lookup(xw, table)
    return out.reshape(B, L, EMB)


# speedup vs baseline: 1.4770x; 1.1303x over previous
import functools

import jax
import jax.numpy as jnp
from jax import lax
from jax.experimental import pallas as pl
from jax.experimental.pallas import tpu as pltpu
from jax.experimental.pallas import tpu_sc as plsc

EMB = 32
B = 4096
L = 200
TOTAL = B * L
NC = 2
NS = 16
NW = NC * NS
PER_W = TOTAL // NW
CHUNK = 1024
NCHUNK = PER_W // CHUNK

_mesh = plsc.VectorSubcoreMesh(core_axis_name="c", subcore_axis_name="s")


@functools.partial(
    pl.kernel,
    out_type=jax.ShapeDtypeStruct((TOTAL, EMB), jnp.float32),
    mesh=_mesh,
    compiler_params=pltpu.CompilerParams(use_tc_tiling_on_sc=False),
    scratch_types=[
        pltpu.VMEM((NCHUNK, CHUNK), jnp.int32),
        pltpu.VMEM((CHUNK, EMB), jnp.float32),
        pltpu.SemaphoreType.DMA,
    ],
)
def _emb_lookup(x_hbm, table_hbm, out_hbm, idx_v, rows_v, sem):
    wid = lax.axis_index("s") * NC + lax.axis_index("c")
    base = wid * PER_W
    pltpu.sync_copy(x_hbm.at[wid], idx_v)

    def body(j, _):
        pltpu.async_copy(table_hbm.at[idx_v.at[j]], rows_v, sem).wait()
        pltpu.sync_copy(rows_v, out_hbm.at[pl.ds(base + j * CHUNK, CHUNK)])
        return 0

    lax.fori_loop(0, NCHUNK, body, 0)


def kernel(x, table):
    xw = x.reshape(NW, NCHUNK, CHUNK).astype(jnp.int32)
    out = _emb_lookup(xw, table)
    return out.reshape(B, L, EMB)


# speedup vs baseline: 1.7061x; 1.1551x over previous
import functools

import jax
import jax.numpy as jnp
from jax import lax
from jax.experimental import pallas as pl
from jax.experimental.pallas import tpu as pltpu
from jax.experimental.pallas import tpu_sc as plsc

EMB = 32
B = 4096
L = 200
TOTAL = B * L
NC = 2
NS = 16
NW = NC * NS
PER_W = TOTAL // NW
CHUNK = 128
NCHUNK = PER_W // CHUNK

_mesh = plsc.VectorSubcoreMesh(core_axis_name="c", subcore_axis_name="s")


@functools.partial(
    pl.kernel,
    out_type=jax.ShapeDtypeStruct((TOTAL, 128), jnp.float32),
    mesh=_mesh,
    compiler_params=pltpu.CompilerParams(use_tc_tiling_on_sc=False),
    scratch_types=[
        pltpu.VMEM((NCHUNK, CHUNK), jnp.int32),
        pltpu.VMEM((CHUNK, EMB), jnp.float32),
        pltpu.SemaphoreType.DMA,
    ],
)
def _emb_lookup(x_hbm, table_hbm, out_hbm, idx_v, rows_v, sem):
    wid = lax.axis_index("s") * NC + lax.axis_index("c")
    base = wid * PER_W
    pltpu.sync_copy(x_hbm.at[wid], idx_v)

    def body(j, _):
        pltpu.async_copy(table_hbm.at[idx_v.at[j]], rows_v, sem).wait()
        pltpu.sync_copy(
            rows_v, out_hbm.at[pl.ds(base + j * CHUNK, CHUNK), pl.ds(0, EMB)]
        )
        return 0

    lax.fori_loop(0, NCHUNK, body, 0)


def kernel(x, table):
    xw = x.reshape(NW, NCHUNK, CHUNK).astype(jnp.int32)
    out = _emb_lookup(xw, table)
    return out[:, :EMB].reshape(B, L, EMB)


# speedup vs baseline: 1.9781x; 1.1594x over previous
import functools

import jax
import jax.numpy as jnp
from jax import lax
from jax.experimental import pallas as pl
from jax.experimental.pallas import tpu as pltpu
from jax.experimental.pallas import tpu_sc as plsc

EMB = 32
B = 4096
L = 200
TOTAL = B * L
NC = 2
NS = 16
NW = NC * NS
BAT_W = B // NW
CHUNK = L
NCHUNK = BAT_W
PER_W = TOTAL // NW

_mesh = plsc.VectorSubcoreMesh(core_axis_name="c", subcore_axis_name="s")


@functools.partial(
    pl.kernel,
    out_type=jax.ShapeDtypeStruct((TOTAL, 128), jnp.float32),
    mesh=_mesh,
    compiler_params=pltpu.CompilerParams(use_tc_tiling_on_sc=False),
    scratch_types=[
        pltpu.VMEM((NCHUNK, CHUNK), jnp.int32),
        pltpu.VMEM((2, CHUNK, EMB), jnp.float32),
        pltpu.SemaphoreType.DMA,
        pltpu.SemaphoreType.DMA,
    ],
)
def _emb_lookup(x_hbm, table_hbm, out_hbm, idx_v, rows_v, gsem, wsem):
    wid = lax.axis_index("s") * NC + lax.axis_index("c")
    base = wid * PER_W
    pltpu.sync_copy(x_hbm.at[pl.ds(wid * BAT_W, BAT_W)], idx_v)

    def gather(j, slot):
        pltpu.async_copy(table_hbm.at[idx_v.at[j]], rows_v.at[slot], gsem)

    def gather_wait(slot):
        pltpu.make_async_copy(
            table_hbm.at[pl.ds(0, CHUNK)], rows_v.at[slot], gsem
        ).wait()

    def write(j, slot):
        pltpu.async_copy(
            rows_v.at[slot],
            out_hbm.at[pl.ds(base + j * CHUNK, CHUNK), pl.ds(0, EMB)],
            wsem,
        )

    def write_wait(j, slot):
        pltpu.make_async_copy(
            rows_v.at[slot],
            out_hbm.at[pl.ds(base + j * CHUNK, CHUNK), pl.ds(0, EMB)],
            wsem,
        ).wait()

    gather(0, 0)

    def body(j, _):
        slot = lax.rem(j, 2)
        nslot = 1 - slot

        @pl.when(j >= 1)
        def _():
            write_wait(j - 1, nslot)

        @pl.when(j + 1 < NCHUNK)
        def _():
            gather(j + 1, nslot)

        gather_wait(slot)
        write(j, slot)
        return 0

    lax.fori_loop(0, NCHUNK, body, 0)
    write_wait(NCHUNK - 1, (NCHUNK - 1) % 2)


def kernel(x, table):
    out = _emb_lookup(x.astype(jnp.int32), table)
    return out[:, :EMB].reshape(B, L, EMB)


# speedup vs baseline: 2.0524x; 1.0376x over previous
import functools

import jax
import jax.numpy as jnp
from jax import lax
from jax.experimental import pallas as pl
from jax.experimental.pallas import tpu as pltpu
from jax.experimental.pallas import tpu_sc as plsc

EMB = 32
B = 4096
L = 200
TOTAL = B * L
NC = 2
NS = 16
NW = NC * NS
PER_W = TOTAL // NW
CHUNK = 1280
NCHUNK = PER_W // CHUNK

_mesh = plsc.VectorSubcoreMesh(core_axis_name="c", subcore_axis_name="s")


@functools.partial(
    pl.kernel,
    out_type=jax.ShapeDtypeStruct((TOTAL, 128), jnp.float32),
    mesh=_mesh,
    compiler_params=pltpu.CompilerParams(use_tc_tiling_on_sc=False),
    scratch_types=[
        pltpu.VMEM((PER_W,), jnp.int32),
        pltpu.VMEM((2, CHUNK, EMB), jnp.float32),
        pltpu.SemaphoreType.DMA,
        pltpu.SemaphoreType.DMA,
    ],
)
def _emb_lookup(x_hbm, table_hbm, out_hbm, idx_v, rows_v, gsem, wsem):
    wid = lax.axis_index("s") * NC + lax.axis_index("c")
    base = wid * PER_W
    pltpu.sync_copy(x_hbm.at[pl.ds(base, PER_W)], idx_v)

    def gather(j, slot):
        pltpu.async_copy(
            table_hbm.at[idx_v.at[pl.ds(j * CHUNK, CHUNK)]], rows_v.at[slot], gsem
        )

    def gather_wait(slot):
        pltpu.make_async_copy(
            table_hbm.at[pl.ds(0, CHUNK)], rows_v.at[slot], gsem
        ).wait()

    def write(j, slot):
        pltpu.async_copy(
            rows_v.at[slot],
            out_hbm.at[pl.ds(base + j * CHUNK, CHUNK), pl.ds(0, EMB)],
            wsem,
        )

    def write_wait(j, slot):
        pltpu.make_async_copy(
            rows_v.at[slot],
            out_hbm.at[pl.ds(base + j * CHUNK, CHUNK), pl.ds(0, EMB)],
            wsem,
        ).wait()

    gather(0, 0)

    def body(j, _):
        slot = lax.rem(j, 2)
        nslot = 1 - slot

        @pl.when(j >= 1)
        def _():
            write_wait(j - 1, nslot)

        @pl.when(j + 1 < NCHUNK)
        def _():
            gather(j + 1, nslot)

        gather_wait(slot)
        write(j, slot)
        return 0

    lax.fori_loop(0, NCHUNK, body, 0)
    write_wait(NCHUNK - 1, (NCHUNK - 1) % 2)


def kernel(x, table):
    out = _emb_lookup(x.reshape(TOTAL).astype(jnp.int32), table)
    return out[:, :EMB].reshape(B, L, EMB)
